# baseline (speedup 1.0000x reference)
import functools

import jax
import jax.numpy as jnp
from jax import lax
from jax.experimental import pallas as pl
from jax.experimental.pallas import tpu as pltpu
from jax.experimental.pallas import tpu_sc as plsc

_B = 16384
_D = 64
_NC = 2
_NS = 16
_NW = _NC * _NS
_BPW = _B // _NW
_L = 16
_H = 256
_NH = _BPW // _H


def _wid_base():
    wid = lax.axis_index("s") * _NC + lax.axis_index("c")
    return wid * _BPW


def _gather_body(row_idx_hbm, row_emb_hbm, out_hbm, ridx, sem):
    base = _wid_base()
    pltpu.sync_copy(row_idx_hbm.at[pl.ds(base, _BPW)], ridx)

    def issue(g, _):
        rv = ridx[pl.ds(g * _L, _L)]
        for k in range(_L):
            j = g * _L + k
            pltpu.async_copy(row_emb_hbm.at[pl.ds(rv[k], 1)],
                             out_hbm.at[pl.ds(base + j, 1)], sem)
        return 0

    lax.fori_loop(0, _BPW // _L, issue, 0)

    def drain(g, _):
        for k in range(_L):
            j = g * _L + k
            pltpu.make_async_copy(row_emb_hbm.at[pl.ds(0, 1)],
                                  out_hbm.at[pl.ds(base + j, 1)], sem).wait()
        return 0

    lax.fori_loop(0, _BPW // _L, drain, 0)


_gather_rows = functools.partial(
    pl.kernel,
    out_type=jax.ShapeDtypeStruct((_B, _D), jnp.float32),
    mesh=plsc.VectorSubcoreMesh(core_axis_name="c", subcore_axis_name="s"),
    scratch_types=[
        pltpu.VMEM((_BPW,), jnp.int32),
        pltpu.SemaphoreType.DMA,
    ],
    compiler_params=pltpu.CompilerParams(use_tc_tiling_on_sc=True),
)(_gather_body)


def _combine_body(col_idx_hbm, dir_idx_hbm, col_emb_hbm, dir_emb_hbm,
                  part_hbm, out_hbm, cidx, didx, rbuf, cbuf, dirv, sem):
    base = _wid_base()

    pltpu.sync_copy(col_idx_hbm.at[pl.ds(base, _BPW)], cidx)
    pltpu.sync_copy(dir_idx_hbm.at[pl.ds(base, _BPW)], didx)
    pltpu.sync_copy(dir_emb_hbm.at[pl.ds(0, 1)], dirv.at[pl.ds(0, 1)])
    pltpu.sync_copy(dir_emb_hbm.at[pl.ds(1, 1)], dirv.at[pl.ds(1, 1)])

    nt = _D // _L
    d0 = [dirv[0, pl.ds(t * _L, _L)] for t in range(nt)]
    dd = [dirv[1, pl.ds(t * _L, _L)] - d0[t] for t in range(nt)]

    for h in range(_NH):
        off = h * _H

        def issue(g, _):
            cv = cidx[pl.ds(off + g * _L, _L)]
            for k in range(_L):
                j = g * _L + k
                pltpu.async_copy(col_emb_hbm.at[pl.ds(cv[k], 1)],
                                 cbuf.at[pl.ds(j, 1)], sem)
            return 0

        lax.fori_loop(0, _H // _L, issue, 0)
        rcp = pltpu.async_copy(part_hbm.at[pl.ds(base + off, _H)], rbuf, sem)

        def drain(g, _):
            for k in range(_L):
                j = g * _L + k
                pltpu.make_async_copy(col_emb_hbm.at[pl.ds(0, 1)],
                                      cbuf.at[pl.ds(j, 1)], sem).wait()
            return 0

        lax.fori_loop(0, _H // _L, drain, 0)
        rcp.wait()

        def combine(g, _):
            fv = didx[pl.ds(off + g * _L, _L)].astype(jnp.float32)
            for k in range(_L):
                b = g * _L + k
                f = fv[k]
                for t in range(nt):
                    s = pl.ds(t * _L, _L)
                    rbuf[b, s] = rbuf[b, s] + cbuf[b, s] + (d0[t] + f * dd[t])
            return 0

        lax.fori_loop(0, _H // _L, combine, 0)
        pltpu.sync_copy(rbuf, out_hbm.at[pl.ds(base + off, _H)])


_combine_rows = functools.partial(
    pl.kernel,
    out_type=jax.ShapeDtypeStruct((_B, _D), jnp.float32),
    mesh=plsc.VectorSubcoreMesh(core_axis_name="c", subcore_axis_name="s"),
    scratch_types=[
        pltpu.VMEM((_BPW,), jnp.int32),
        pltpu.VMEM((_BPW,), jnp.int32),
        pltpu.VMEM((_H, _D), jnp.float32),
        pltpu.VMEM((_H, _D), jnp.float32),
        pltpu.VMEM((2, _D), jnp.float32),
        pltpu.SemaphoreType.DMA,
    ],
    compiler_params=pltpu.CompilerParams(use_tc_tiling_on_sc=True),
)(_combine_body)


def kernel(row_idx, col_idx, dir_idx, row_emb, col_emb, dir_emb):
    ri = row_idx.astype(jnp.int32)
    ci = col_idx.astype(jnp.int32)
    di = dir_idx.astype(jnp.int32)
    part = _gather_rows(ri, row_emb)
    return _combine_rows(ci, di, col_emb, dir_emb, part)


# speedup vs baseline: 2.7658x; 2.7658x over previous
import functools

import jax
import jax.numpy as jnp
from jax import lax
from jax.experimental import pallas as pl
from jax.experimental.pallas import tpu as pltpu
from jax.experimental.pallas import tpu_sc as plsc

_B = 16384
_D = 64
_NC = 2
_NS = 16
_NW = _NC * _NS
_BPW = _B // _NW
_L = 16
_H = 256
_NH = _BPW // _H


def _wid_base():
    wid = lax.axis_index("s") * _NC + lax.axis_index("c")
    return wid * _BPW


def _gather_body(row_idx_hbm, row_emb_hbm, out_hbm, ridx, rbuf, sem):
    base = _wid_base()
    pltpu.sync_copy(row_idx_hbm.at[pl.ds(base, _BPW)], ridx)

    for h in range(_NH):
        off = h * _H

        def issue(g, _):
            rv = ridx[pl.ds(off + g * _L, _L)]
            for k in range(_L):
                j = g * _L + k
                pltpu.async_copy(row_emb_hbm.at[pl.ds(rv[k], 1)],
                                 rbuf.at[pl.ds(j, 1)], sem)
            return 0

        lax.fori_loop(0, _H // _L, issue, 0)

        def drain(g, _):
            for k in range(_L):
                j = g * _L + k
                pltpu.make_async_copy(row_emb_hbm.at[pl.ds(0, 1)],
                                      rbuf.at[pl.ds(j, 1)], sem).wait()
            return 0

        lax.fori_loop(0, _H // _L, drain, 0)
        pltpu.sync_copy(rbuf, out_hbm.at[pl.ds(base + off, _H)])


_gather_rows = functools.partial(
    pl.kernel,
    out_type=jax.ShapeDtypeStruct((_B, _D), jnp.float32),
    mesh=plsc.VectorSubcoreMesh(core_axis_name="c", subcore_axis_name="s"),
    scratch_types=[
        pltpu.VMEM((_BPW,), jnp.int32),
        pltpu.VMEM((_H, _D), jnp.float32),
        pltpu.SemaphoreType.DMA,
    ],
    compiler_params=pltpu.CompilerParams(use_tc_tiling_on_sc=True),
)(_gather_body)


def _combine_body(col_idx_hbm, dir_idx_hbm, col_emb_hbm, dir_emb_hbm,
                  part_hbm, out_hbm, cidx, didx, rbuf, cbuf, dirv, sem):
    base = _wid_base()

    pltpu.sync_copy(col_idx_hbm.at[pl.ds(base, _BPW)], cidx)
    pltpu.sync_copy(dir_idx_hbm.at[pl.ds(base, _BPW)], didx)
    pltpu.sync_copy(dir_emb_hbm.at[pl.ds(0, 1)], dirv.at[pl.ds(0, 1)])
    pltpu.sync_copy(dir_emb_hbm.at[pl.ds(1, 1)], dirv.at[pl.ds(1, 1)])

    nt = _D // _L
    d0 = [dirv[0, pl.ds(t * _L, _L)] for t in range(nt)]
    dd = [dirv[1, pl.ds(t * _L, _L)] - d0[t] for t in range(nt)]

    for h in range(_NH):
        off = h * _H

        def issue(g, _):
            cv = cidx[pl.ds(off + g * _L, _L)]
            for k in range(_L):
                j = g * _L + k
                pltpu.async_copy(col_emb_hbm.at[pl.ds(cv[k], 1)],
                                 cbuf.at[pl.ds(j, 1)], sem)
            return 0

        lax.fori_loop(0, _H // _L, issue, 0)
        rcp = pltpu.async_copy(part_hbm.at[pl.ds(base + off, _H)], rbuf, sem)

        def drain(g, _):
            for k in range(_L):
                j = g * _L + k
                pltpu.make_async_copy(col_emb_hbm.at[pl.ds(0, 1)],
                                      cbuf.at[pl.ds(j, 1)], sem).wait()
            return 0

        lax.fori_loop(0, _H // _L, drain, 0)
        rcp.wait()

        def combine(g, _):
            fv = didx[pl.ds(off + g * _L, _L)].astype(jnp.float32)
            for k in range(_L):
                b = g * _L + k
                f = fv[k]
                for t in range(nt):
                    s = pl.ds(t * _L, _L)
                    rbuf[b, s] = rbuf[b, s] + cbuf[b, s] + (d0[t] + f * dd[t])
            return 0

        lax.fori_loop(0, _H // _L, combine, 0)
        pltpu.sync_copy(rbuf, out_hbm.at[pl.ds(base + off, _H)])


_combine_rows = functools.partial(
    pl.kernel,
    out_type=jax.ShapeDtypeStruct((_B, _D), jnp.float32),
    mesh=plsc.VectorSubcoreMesh(core_axis_name="c", subcore_axis_name="s"),
    scratch_types=[
        pltpu.VMEM((_BPW,), jnp.int32),
        pltpu.VMEM((_BPW,), jnp.int32),
        pltpu.VMEM((_H, _D), jnp.float32),
        pltpu.VMEM((_H, _D), jnp.float32),
        pltpu.VMEM((2, _D), jnp.float32),
        pltpu.SemaphoreType.DMA,
    ],
    compiler_params=pltpu.CompilerParams(use_tc_tiling_on_sc=True),
)(_combine_body)


def kernel(row_idx, col_idx, dir_idx, row_emb, col_emb, dir_emb):
    ri = row_idx.astype(jnp.int32)
    ci = col_idx.astype(jnp.int32)
    di = dir_idx.astype(jnp.int32)
    part = _gather_rows(ri, row_emb)
    return _combine_rows(ci, di, col_emb, dir_emb, part)


# speedup vs baseline: 3.0346x; 1.0972x over previous
import functools

import jax
import jax.numpy as jnp
from jax import lax
from jax.experimental import pallas as pl
from jax.experimental.pallas import tpu as pltpu
from jax.experimental.pallas import tpu_sc as plsc

_B = 16384
_D = 64
_NC = 2
_NS = 16
_NW = _NC * _NS
_BPW = _B // _NW
_L = 16
_H = 256
_NH = _BPW // _H


def _wid_base():
    wid = lax.axis_index("s") * _NC + lax.axis_index("c")
    return wid * _BPW


def _gather_body(row_idx_hbm, dir_idx_hbm, row_emb_hbm, dir_emb_hbm,
                 out_hbm, ridx, didx, rbuf, dirv, sem):
    base = _wid_base()
    pltpu.sync_copy(row_idx_hbm.at[pl.ds(base, _BPW)], ridx)
    pltpu.sync_copy(dir_idx_hbm.at[pl.ds(base, _BPW)], didx)
    pltpu.sync_copy(dir_emb_hbm.at[pl.ds(0, 1)], dirv.at[pl.ds(0, 1)])
    pltpu.sync_copy(dir_emb_hbm.at[pl.ds(1, 1)], dirv.at[pl.ds(1, 1)])

    nt = _D // _L
    d0 = [dirv[0, pl.ds(t * _L, _L)] for t in range(nt)]
    dd = [dirv[1, pl.ds(t * _L, _L)] - d0[t] for t in range(nt)]

    for h in range(_NH):
        off = h * _H

        def issue(g, _):
            rv = ridx[pl.ds(off + g * _L, _L)]
            for k in range(_L):
                j = g * _L + k
                pltpu.async_copy(row_emb_hbm.at[pl.ds(rv[k], 1)],
                                 rbuf.at[pl.ds(j, 1)], sem)
            return 0

        lax.fori_loop(0, _H // _L, issue, 0)

        def drain(g, _):
            for k in range(_L):
                j = g * _L + k
                pltpu.make_async_copy(row_emb_hbm.at[pl.ds(0, 1)],
                                      rbuf.at[pl.ds(j, 1)], sem).wait()
            return 0

        lax.fori_loop(0, _H // _L, drain, 0)

        def adddir(g, _):
            fv = didx[pl.ds(off + g * _L, _L)].astype(jnp.float32)
            for k in range(_L):
                b = g * _L + k
                f = fv[k]
                for t in range(nt):
                    s = pl.ds(t * _L, _L)
                    rbuf[b, s] = rbuf[b, s] + (d0[t] + f * dd[t])
            return 0

        lax.fori_loop(0, _H // _L, adddir, 0)
        pltpu.sync_copy(rbuf, out_hbm.at[pl.ds(base + off, _H)])


_gather_rows = functools.partial(
    pl.kernel,
    out_type=jax.ShapeDtypeStruct((_B, _D), jnp.float32),
    mesh=plsc.VectorSubcoreMesh(core_axis_name="c", subcore_axis_name="s"),
    scratch_types=[
        pltpu.VMEM((_BPW,), jnp.int32),
        pltpu.VMEM((_BPW,), jnp.int32),
        pltpu.VMEM((_H, _D), jnp.float32),
        pltpu.VMEM((2, _D), jnp.float32),
        pltpu.SemaphoreType.DMA,
    ],
    compiler_params=pltpu.CompilerParams(use_tc_tiling_on_sc=True),
)(_gather_body)


def _combine_body(col_idx_hbm, col_emb_hbm, part_hbm, out_hbm,
                  cidx, rbuf, cbuf, sem):
    base = _wid_base()

    pltpu.sync_copy(col_idx_hbm.at[pl.ds(base, _BPW)], cidx)

    nt = _D // _L
    for h in range(_NH):
        off = h * _H

        def issue(g, _):
            cv = cidx[pl.ds(off + g * _L, _L)]
            for k in range(_L):
                j = g * _L + k
                pltpu.async_copy(col_emb_hbm.at[pl.ds(cv[k], 1)],
                                 cbuf.at[pl.ds(j, 1)], sem)
            return 0

        lax.fori_loop(0, _H // _L, issue, 0)
        rcp = pltpu.async_copy(part_hbm.at[pl.ds(base + off, _H)], rbuf, sem)

        def drain(g, _):
            for k in range(_L):
                j = g * _L + k
                pltpu.make_async_copy(col_emb_hbm.at[pl.ds(0, 1)],
                                      cbuf.at[pl.ds(j, 1)], sem).wait()
            return 0

        lax.fori_loop(0, _H // _L, drain, 0)
        rcp.wait()

        def combine(g, _):
            for k in range(_L):
                b = g * _L + k
                for t in range(nt):
                    s = pl.ds(t * _L, _L)
                    rbuf[b, s] = rbuf[b, s] + cbuf[b, s]
            return 0

        lax.fori_loop(0, _H // _L, combine, 0)
        pltpu.sync_copy(rbuf, out_hbm.at[pl.ds(base + off, _H)])


_combine_rows = functools.partial(
    pl.kernel,
    out_type=jax.ShapeDtypeStruct((_B, _D), jnp.float32),
    mesh=plsc.VectorSubcoreMesh(core_axis_name="c", subcore_axis_name="s"),
    scratch_types=[
        pltpu.VMEM((_BPW,), jnp.int32),
        pltpu.VMEM((_H, _D), jnp.float32),
        pltpu.VMEM((_H, _D), jnp.float32),
        pltpu.SemaphoreType.DMA,
    ],
    compiler_params=pltpu.CompilerParams(use_tc_tiling_on_sc=True),
)(_combine_body)


def kernel(row_idx, col_idx, dir_idx, row_emb, col_emb, dir_emb):
    ri = row_idx.astype(jnp.int32)
    ci = col_idx.astype(jnp.int32)
    di = dir_idx.astype(jnp.int32)
    part = _gather_rows(ri, di, row_emb, dir_emb)
    return _combine_rows(ci, col_emb, part)
